# baseline (speedup 1.0000x reference)
import functools

import jax
import jax.numpy as jnp
from jax import lax
from jax.experimental import pallas as pl
from jax.experimental.pallas import tpu as pltpu
from jax.experimental.pallas import tpu_sc as plsc

N = 10000
E = 320000
D = 128
G = 128
C = 10

NC = 2
NS = 16
NW = NC * NS

CHUNK = 64
TPC = 160
CHUNKS_PAD = NW * TPC
E_PAD = CHUNKS_PAD * CHUNK
PH = 16
NPH = TPC // PH
PAIRS = PH // 2
RPT = 640
N_ACC = NS * RPT
NBLOCK = RPT // CHUNK

_sc_mesh = plsc.VectorSubcoreMesh(
    core_axis_name="c", subcore_axis_name="s", num_cores=NC, num_subcores=NS)


@functools.partial(
    pl.kernel,
    out_type=[
        jax.ShapeDtypeStruct((NC * N_ACC, D), jnp.float32),
        jax.ShapeDtypeStruct((NC * N_ACC, D), jnp.float32),
    ],
    mesh=_sc_mesh,
    compiler_params=pltpu.CompilerParams(use_tc_tiling_on_sc=False),
    scratch_types=[
        pltpu.VMEM((PH, CHUNK), jnp.int32),
        pltpu.VMEM((PH, CHUNK), jnp.int32),
        pltpu.VMEM((PH, CHUNK), jnp.int32),
        pltpu.VMEM((PH, CHUNK), jnp.int32),
        pltpu.VMEM((CHUNK, D), jnp.float32),
        pltpu.VMEM((CHUNK, D), jnp.float32),
        pltpu.VMEM_SHARED((N_ACC, D), jnp.float32),
        pltpu.SemaphoreType.DMA,
        pltpu.SemaphoreType.DMA,
        pltpu.SemaphoreType.DMA,
        pltpu.SemaphoreType.DMA,
        pltpu.SemaphoreType.DMA,
        pltpu.SemaphoreType.DMA,
    ],
)
def _sc_agg(src_hbm, dst_hbm, x_hbm, ones_hbm, z_hbm,
            agg_out, deg_out,
            idx_s0, idx_s1, idx_d0, idx_d1, rows_a, rows_b, acc_sh,
            semg0, semg1, sema, semb, semss, semsd):
    c = lax.axis_index("c")
    s = lax.axis_index("s")
    wid = c * NS + s
    start = wid * TPC
    out_base = c * N_ACC + s * RPT

    idx_s = (idx_s0, idx_s1)
    idx_d = (idx_d0, idx_d1)

    pltpu.sync_copy(z_hbm, rows_b)
    for j in range(NBLOCK):
        pltpu.sync_copy(rows_b, acc_sh.at[pl.ds(s * RPT + j * CHUNK, CHUNK)])
    pltpu.sync_copy(src_hbm.at[pl.ds(start, PH)], idx_s0)
    pltpu.sync_copy(dst_hbm.at[pl.ds(start, PH)], idx_d0)
    plsc.subcore_barrier()

    for p in range(NPH):
        cur_s, cur_d = idx_s[p % 2], idx_d[p % 2]
        hs = hd = None
        if p + 1 < NPH:
            nxt = pl.ds(start + (p + 1) * PH, PH)
            hs = pltpu.async_copy(src_hbm.at[nxt], idx_s[(p + 1) % 2], semss)
            hd = pltpu.async_copy(dst_hbm.at[nxt], idx_d[(p + 1) % 2], semsd)

        def body(j, carry, cur_s=cur_s, cur_d=cur_d):
            g0 = pltpu.async_copy(x_hbm.at[cur_s.at[2 * j]], rows_a, semg0)
            g1 = pltpu.async_copy(x_hbm.at[cur_s.at[2 * j + 1]], rows_b, semg1)
            g0.wait()
            s0 = pltpu.async_copy(rows_a, acc_sh.at[cur_d.at[2 * j]], sema,
                                  add=True)
            g1.wait()
            s1 = pltpu.async_copy(rows_b, acc_sh.at[cur_d.at[2 * j + 1]], semb,
                                  add=True)
            s0.wait()
            s1.wait()
            return carry

        lax.fori_loop(0, PAIRS, body, 0)
        if hs is not None:
            hs.wait()
            hd.wait()
    plsc.subcore_barrier()

    for j in range(NBLOCK):
        roff = s * RPT + j * CHUNK
        pltpu.sync_copy(acc_sh.at[pl.ds(roff, CHUNK)], rows_a)
        pltpu.sync_copy(rows_a, agg_out.at[pl.ds(out_base + j * CHUNK, CHUNK)])
    pltpu.sync_copy(z_hbm, rows_b)
    for j in range(NBLOCK):
        pltpu.sync_copy(rows_b, acc_sh.at[pl.ds(s * RPT + j * CHUNK, CHUNK)])
    pltpu.sync_copy(ones_hbm, rows_a)
    pltpu.sync_copy(dst_hbm.at[pl.ds(start, PH)], idx_d0)
    plsc.subcore_barrier()

    for p in range(NPH):
        cur_d = idx_d[p % 2]
        hd = None
        if p + 1 < NPH:
            nxt = pl.ds(start + (p + 1) * PH, PH)
            hd = pltpu.async_copy(dst_hbm.at[nxt], idx_d[(p + 1) % 2], semsd)

        def body2(j, carry, cur_d=cur_d):
            s0 = pltpu.async_copy(rows_a, acc_sh.at[cur_d.at[2 * j]], sema,
                                  add=True)
            s1 = pltpu.async_copy(rows_a, acc_sh.at[cur_d.at[2 * j + 1]], semb,
                                  add=True)
            s0.wait()
            s1.wait()
            return carry

        lax.fori_loop(0, PAIRS, body2, 0)
        if hd is not None:
            hd.wait()
    plsc.subcore_barrier()

    for j in range(NBLOCK):
        roff = s * RPT + j * CHUNK
        pltpu.sync_copy(acc_sh.at[pl.ds(roff, CHUNK)], rows_b)
        pltpu.sync_copy(rows_b, deg_out.at[pl.ds(out_base + j * CHUNK, CHUNK)])


RB = 400
NBLK = N // RB


def _tc_post_body(agg_ref, deg_ref, batch_ref, w1_ref, b1_ref, w2_ref, b2_ref,
                  out_ref, pooled_ref):
    i = pl.program_id(0)

    agg = agg_ref[0] + agg_ref[1]
    deg = deg_ref[0, :, 0:1] + deg_ref[1, :, 0:1]
    xm = agg / jnp.maximum(deg, 1.0)
    h = jnp.dot(xm, w1_ref[...], preferred_element_type=jnp.float32)
    h = jnp.maximum(h + b1_ref[...], 0.0)

    b = batch_ref[0]
    gids = lax.broadcasted_iota(jnp.int32, (G, 1), 0)
    oh = (b == gids).astype(jnp.float32)

    @pl.when(i == 0)
    def _():
        pooled_ref[...] = jnp.zeros_like(pooled_ref)

    pooled_ref[...] += jnp.dot(oh, h, preferred_element_type=jnp.float32)

    @pl.when(i == NBLK - 1)
    def _():
        out_ref[...] = (
            jnp.dot(pooled_ref[...], w2_ref[...],
                    preferred_element_type=jnp.float32) + b2_ref[...])


_tc_post = pl.pallas_call(
    _tc_post_body,
    grid=(NBLK,),
    in_specs=[
        pl.BlockSpec((NC, RB, D), lambda i: (0, i, 0)),
        pl.BlockSpec((NC, RB, D), lambda i: (0, i, 0)),
        pl.BlockSpec((1, 1, RB), lambda i: (i, 0, 0)),
        pl.BlockSpec((D, D), lambda i: (0, 0)),
        pl.BlockSpec((1, D), lambda i: (0, 0)),
        pl.BlockSpec((D, C), lambda i: (0, 0)),
        pl.BlockSpec((1, C), lambda i: (0, 0)),
    ],
    out_specs=pl.BlockSpec((G, C), lambda i: (0, 0)),
    out_shape=jax.ShapeDtypeStruct((G, C), jnp.float32),
    scratch_shapes=[pltpu.VMEM((G, D), jnp.float32)],
)


@jax.jit
def kernel(x, edge_index, batch, W1, b1, W2, b2):
    npad = E_PAD - E
    src2d = jnp.concatenate(
        [edge_index[0], jnp.zeros((npad,), jnp.int32)]).reshape(CHUNKS_PAD, CHUNK)
    dst2d = jnp.concatenate(
        [edge_index[1], jnp.full((npad,), N, jnp.int32)]).reshape(CHUNKS_PAD, CHUNK)
    ones = jnp.ones((CHUNK, D), jnp.float32)
    z128 = jnp.zeros((CHUNK, D), jnp.float32)
    agg2, deg2 = _sc_agg(src2d, dst2d, x, ones, z128)
    agg3 = agg2.reshape(NC, N_ACC, D)
    deg3 = deg2.reshape(NC, N_ACC, D)
    batch3d = batch.reshape(NBLK, 1, RB)
    return _tc_post(agg3, deg3, batch3d, W1, b1.reshape(1, D),
                    W2, b2.reshape(1, C))


# speedup vs baseline: 1.0003x; 1.0003x over previous
import functools

import jax
import jax.numpy as jnp
from jax import lax
from jax.experimental import pallas as pl
from jax.experimental.pallas import tpu as pltpu
from jax.experimental.pallas import tpu_sc as plsc

N = 10000
E = 320000
D = 128
G = 128
C = 10

NC = 2
NS = 16
NW = NC * NS

CHUNK = 128
TPC = 80
CHUNKS_PAD = NW * TPC
E_PAD = CHUNKS_PAD * CHUNK
PH = 8
NPH = TPC // PH
PAIRS = PH // 2
RPT = 640
N_ACC = NS * RPT
NBLOCK = RPT // CHUNK

_sc_mesh = plsc.VectorSubcoreMesh(
    core_axis_name="c", subcore_axis_name="s", num_cores=NC, num_subcores=NS)


@functools.partial(
    pl.kernel,
    out_type=[
        jax.ShapeDtypeStruct((NC * N_ACC, D), jnp.float32),
        jax.ShapeDtypeStruct((NC * N_ACC, D), jnp.float32),
    ],
    mesh=_sc_mesh,
    compiler_params=pltpu.CompilerParams(use_tc_tiling_on_sc=False),
    scratch_types=[
        pltpu.VMEM((PH, CHUNK), jnp.int32),
        pltpu.VMEM((PH, CHUNK), jnp.int32),
        pltpu.VMEM((PH, CHUNK), jnp.int32),
        pltpu.VMEM((PH, CHUNK), jnp.int32),
        pltpu.VMEM((CHUNK, D), jnp.float32),
        pltpu.VMEM_SHARED((N_ACC, D), jnp.float32),
        pltpu.SemaphoreType.DMA,
        pltpu.SemaphoreType.DMA,
        pltpu.SemaphoreType.DMA,
        pltpu.SemaphoreType.DMA,
        pltpu.SemaphoreType.DMA,
    ],
)
def _sc_agg(src_hbm, dst_hbm, x_hbm, ones_hbm, z_hbm,
            agg_out, deg_out,
            idx_s0, idx_s1, idx_d0, idx_d1, rows_v, acc_sh,
            semg, sema, semb, semss, semsd):
    c = lax.axis_index("c")
    s = lax.axis_index("s")
    wid = c * NS + s
    start = wid * TPC
    out_base = c * N_ACC + s * RPT

    idx_s = (idx_s0, idx_s1)
    idx_d = (idx_d0, idx_d1)

    pltpu.sync_copy(z_hbm, rows_v)
    for j in range(NBLOCK):
        pltpu.sync_copy(rows_v, acc_sh.at[pl.ds(s * RPT + j * CHUNK, CHUNK)])
    pltpu.sync_copy(src_hbm.at[pl.ds(start, PH)], idx_s0)
    pltpu.sync_copy(dst_hbm.at[pl.ds(start, PH)], idx_d0)
    plsc.subcore_barrier()

    for p in range(NPH):
        cur_s, cur_d = idx_s[p % 2], idx_d[p % 2]
        hs = hd = None
        if p + 1 < NPH:
            nxt = pl.ds(start + (p + 1) * PH, PH)
            hs = pltpu.async_copy(src_hbm.at[nxt], idx_s[(p + 1) % 2], semss)
            hd = pltpu.async_copy(dst_hbm.at[nxt], idx_d[(p + 1) % 2], semsd)

        def body(t, carry, cur_s=cur_s, cur_d=cur_d):
            pltpu.async_copy(x_hbm.at[cur_s.at[t]], rows_v, semg).wait()
            pltpu.async_copy(rows_v, acc_sh.at[cur_d.at[t]], sema,
                             add=True).wait()
            return carry

        lax.fori_loop(0, PH, body, 0)
        if hs is not None:
            hs.wait()
            hd.wait()
    plsc.subcore_barrier()

    for j in range(NBLOCK):
        roff = s * RPT + j * CHUNK
        pltpu.sync_copy(acc_sh.at[pl.ds(roff, CHUNK)], rows_v)
        pltpu.sync_copy(rows_v, agg_out.at[pl.ds(out_base + j * CHUNK, CHUNK)])
    pltpu.sync_copy(z_hbm, rows_v)
    for j in range(NBLOCK):
        pltpu.sync_copy(rows_v, acc_sh.at[pl.ds(s * RPT + j * CHUNK, CHUNK)])
    pltpu.sync_copy(ones_hbm, rows_v)
    pltpu.sync_copy(dst_hbm.at[pl.ds(start, PH)], idx_d0)
    plsc.subcore_barrier()

    for p in range(NPH):
        cur_d = idx_d[p % 2]
        hd = None
        if p + 1 < NPH:
            nxt = pl.ds(start + (p + 1) * PH, PH)
            hd = pltpu.async_copy(dst_hbm.at[nxt], idx_d[(p + 1) % 2], semsd)

        def body2(j, carry, cur_d=cur_d):
            s0 = pltpu.async_copy(rows_v, acc_sh.at[cur_d.at[2 * j]], sema,
                                  add=True)
            s1 = pltpu.async_copy(rows_v, acc_sh.at[cur_d.at[2 * j + 1]], semb,
                                  add=True)
            s0.wait()
            s1.wait()
            return carry

        lax.fori_loop(0, PAIRS, body2, 0)
        if hd is not None:
            hd.wait()
    plsc.subcore_barrier()

    for j in range(NBLOCK):
        roff = s * RPT + j * CHUNK
        pltpu.sync_copy(acc_sh.at[pl.ds(roff, CHUNK)], rows_v)
        pltpu.sync_copy(rows_v, deg_out.at[pl.ds(out_base + j * CHUNK, CHUNK)])


RB = 400
NBLK = N // RB


def _tc_post_body(agg_ref, deg_ref, batch_ref, w1_ref, b1_ref, w2_ref, b2_ref,
                  out_ref, pooled_ref):
    i = pl.program_id(0)

    agg = agg_ref[0] + agg_ref[1]
    deg = deg_ref[0, :, 0:1] + deg_ref[1, :, 0:1]
    xm = agg / jnp.maximum(deg, 1.0)
    h = jnp.dot(xm, w1_ref[...], preferred_element_type=jnp.float32)
    h = jnp.maximum(h + b1_ref[...], 0.0)

    b = batch_ref[0]
    gids = lax.broadcasted_iota(jnp.int32, (G, 1), 0)
    oh = (b == gids).astype(jnp.float32)

    @pl.when(i == 0)
    def _():
        pooled_ref[...] = jnp.zeros_like(pooled_ref)

    pooled_ref[...] += jnp.dot(oh, h, preferred_element_type=jnp.float32)

    @pl.when(i == NBLK - 1)
    def _():
        out_ref[...] = (
            jnp.dot(pooled_ref[...], w2_ref[...],
                    preferred_element_type=jnp.float32) + b2_ref[...])


_tc_post = pl.pallas_call(
    _tc_post_body,
    grid=(NBLK,),
    in_specs=[
        pl.BlockSpec((NC, RB, D), lambda i: (0, i, 0)),
        pl.BlockSpec((NC, RB, D), lambda i: (0, i, 0)),
        pl.BlockSpec((1, 1, RB), lambda i: (i, 0, 0)),
        pl.BlockSpec((D, D), lambda i: (0, 0)),
        pl.BlockSpec((1, D), lambda i: (0, 0)),
        pl.BlockSpec((D, C), lambda i: (0, 0)),
        pl.BlockSpec((1, C), lambda i: (0, 0)),
    ],
    out_specs=pl.BlockSpec((G, C), lambda i: (0, 0)),
    out_shape=jax.ShapeDtypeStruct((G, C), jnp.float32),
    scratch_shapes=[pltpu.VMEM((G, D), jnp.float32)],
)


@jax.jit
def kernel(x, edge_index, batch, W1, b1, W2, b2):
    npad = E_PAD - E
    src2d = jnp.concatenate(
        [edge_index[0], jnp.zeros((npad,), jnp.int32)]).reshape(CHUNKS_PAD, CHUNK)
    dst2d = jnp.concatenate(
        [edge_index[1], jnp.full((npad,), N, jnp.int32)]).reshape(CHUNKS_PAD, CHUNK)
    ones = jnp.ones((CHUNK, D), jnp.float32)
    z128 = jnp.zeros((CHUNK, D), jnp.float32)
    agg2, deg2 = _sc_agg(src2d, dst2d, x, ones, z128)
    agg3 = agg2.reshape(NC, N_ACC, D)
    deg3 = deg2.reshape(NC, N_ACC, D)
    batch3d = batch.reshape(NBLK, 1, RB)
    return _tc_post(agg3, deg3, batch3d, W1, b1.reshape(1, D),
                    W2, b2.reshape(1, C))
